# baseline (speedup 1.0000x reference)
import functools

import jax
import jax.numpy as jnp
from jax import lax
from jax.experimental import pallas as pl
from jax.experimental.pallas import tpu as pltpu

BS, CH, T_IN, T_OUT, EXP_DIM, TOP_K, NCLS = 4096, 64, 128, 64, 32, 2, 3
CH_HALF = CH // 2

B_TILE = 32
NTB = B_TILE * CH_HALF
E_CHUNK = 16


def _ln(x, g, b, eps=1e-5):
    mu = jnp.mean(x, axis=-1, keepdims=True)
    xc = x - mu
    var = jnp.mean(xc * xc, axis=-1, keepdims=True)
    return xc / jnp.sqrt(var + eps) * g + b


def _bdot(a, b):
    return jnp.dot(a.astype(jnp.bfloat16), b.astype(jnp.bfloat16),
                   preferred_element_type=jnp.float32)


def _body(xl_ref, xr_ref, W0_ref, W1_ref, b0_ref, b1_ref, bn_g_ref, bn_b_ref,
          centers_ref, proj_W_ref, proj_b_ref, expWt_ref, exp_b_ref,
          moe_lg_ref, moe_lb_ref, moe_rg_ref, moe_rb_ref,
          qW_ref, qb_ref, kW_ref, kb_ref, vW_ref, vb_ref, oW_ref, ob_ref,
          att_lg_ref, att_lb_ref, att_rg_ref, att_rb_ref,
          headWl_ref, headWr_ref, headb_ref, mask_ref, out_ref):
    f32 = jnp.float32

    W0 = W0_ref[...]
    W1 = W1_ref[...]
    b0 = b0_ref[...]
    b1 = b1_ref[...]
    bn_g = bn_g_ref[...]
    bn_b = bn_b_ref[...]
    xl = xl_ref[...]
    xr = xr_ref[...]
    hl = jnp.maximum((_bdot(xl, W0) - b0) + (_bdot(xl, W1) - b1), 0.0)
    hr = jnp.maximum((_bdot(xr, W0) - b0) + (_bdot(xr, W1) - b1), 0.0)
    g_l = _ln(hl, bn_g, bn_b)
    g_r = _ln(hr, bn_g, bn_b)

    xcat = jnp.concatenate([g_l, g_r], axis=1)
    pr = _bdot(xcat, proj_W_ref[...]) + proj_b_ref[...]
    xp = pr / jnp.maximum(jnp.sqrt(jnp.sum(pr * pr, axis=-1, keepdims=True)),
                          1e-12)
    cn = centers_ref[...]
    cn = cn / jnp.maximum(jnp.sqrt(jnp.sum(cn * cn, axis=-1, keepdims=True)),
                          1e-12)
    sim = lax.dot_general(xp.astype(jnp.bfloat16), cn.astype(jnp.bfloat16),
                          (((1,), (1,)), ((), ())),
                          preferred_element_type=f32)

    eidx = lax.broadcasted_iota(jnp.int32, sim.shape, 1)
    m1 = jnp.max(sim, axis=-1, keepdims=True)
    i1 = jnp.min(jnp.where(sim == m1, eidx, CH), axis=-1, keepdims=True)
    sim2 = jnp.where(eidx == i1, -jnp.inf, sim)
    m2 = jnp.max(sim2, axis=-1, keepdims=True)
    i2 = jnp.min(jnp.where(sim2 == m2, eidx, CH), axis=-1, keepdims=True)
    w = (jnp.where(eidx == i1, 1.0, 0.0)
         + jnp.where(eidx == i2, 1.0, 0.0))

    gb_l = g_l.astype(jnp.bfloat16)
    gb_r = g_r.astype(jnp.bfloat16)
    Wt = expWt_ref[...].astype(jnp.bfloat16)
    acc_l = jnp.dot(w, exp_b_ref[...], preferred_element_type=f32)
    acc_r = acc_l
    for ec in range(0, CH, E_CHUNK):
        Wc = Wt[:, ec * T_OUT:(ec + E_CHUNK) * T_OUT]
        Pl = jnp.dot(gb_l, Wc, preferred_element_type=f32)
        Pr = jnp.dot(gb_r, Wc, preferred_element_type=f32)
        for j in range(E_CHUNK):
            e = ec + j
            we = w[:, e:e + 1]
            acc_l = acc_l + we * Pl[:, j * T_OUT:(j + 1) * T_OUT]
            acc_r = acc_r + we * Pr[:, j * T_OUT:(j + 1) * T_OUT]
    m_l = _ln(acc_l, moe_lg_ref[...], moe_lb_ref[...]) + g_l
    m_r = _ln(acc_r, moe_rg_ref[...], moe_rb_ref[...]) + g_r

    q = (_bdot(m_l, qW_ref[...]) + qb_ref[...]).astype(jnp.bfloat16)
    k = (_bdot(m_r, kW_ref[...]) + kb_ref[...]).astype(jnp.bfloat16)
    v = (_bdot(m_l - m_r, vW_ref[...]) + vb_ref[...]).astype(jnp.bfloat16)
    energy = lax.dot_general(q, k, (((1,), (1,)), ((), ())),
                             preferred_element_type=f32)
    energy = energy * (1.0 / (T_OUT ** 0.5)) + mask_ref[...]
    emax = jnp.max(energy, axis=-1, keepdims=True)
    ee = jnp.exp(energy - emax)
    attn = (ee / jnp.sum(ee, axis=-1, keepdims=True)).astype(jnp.bfloat16)
    o_l = jnp.dot(attn, v, preferred_element_type=f32)
    o_r = lax.dot_general(attn, v, (((0,), (0,)), ((), ())),
                          preferred_element_type=f32)
    oW = oW_ref[...]
    ob = ob_ref[...]
    f_l = _ln(_bdot(o_l, oW) + ob, att_lg_ref[...], att_lb_ref[...]) + m_l
    f_r = _ln(_bdot(o_r, oW) + ob, att_rg_ref[...], att_rb_ref[...]) + m_r

    fl3 = f_l.reshape(B_TILE, CH_HALF, T_OUT)
    fr3 = f_r.reshape(B_TILE, CH_HALF, T_OUT)
    Hl = headWl_ref[...]
    Hr = headWr_ref[...]
    cols = []
    for j in range(NCLS):
        s = (jnp.sum(fl3 * Hl[:, :, j][None], axis=(1, 2))
             + jnp.sum(fr3 * Hr[:, :, j][None], axis=(1, 2)))
        cols.append(s[:, None])
    out_ref[...] = jnp.concatenate(cols, axis=1) + headb_ref[...]


@jax.jit
def kernel(x_l, x_r, A, W0, b0, W1, b1, bn_g, bn_b, centers, proj_W, proj_b,
           exp_W, exp_b, moe_lg, moe_lb, moe_rg, moe_rb, qW, qb, kW, kb,
           vW, vb, oW, ob, att_lg, att_lb, att_rg, att_rb, headW, headb):
    del A
    xl2 = x_l.reshape(BS * CH_HALF, T_IN)
    xr2 = x_r.reshape(BS * CH_HALF, T_IN)
    b0r = b0.reshape(1, T_OUT)
    b1r = b1.reshape(1, T_OUT)
    expWt = exp_W.transpose(1, 0, 2).reshape(T_OUT, CH * T_OUT)
    headWl = headW[:CH_HALF * T_OUT].reshape(CH_HALF, T_OUT, NCLS)
    headWr = headW[CH_HALF * T_OUT:].reshape(CH_HALF, T_OUT, NCLS)
    row2 = lambda a: a.reshape(1, -1)

    grid = (BS // B_TILE,)
    full = lambda a: pl.BlockSpec(a.shape, lambda i: (0,) * a.ndim)
    in_specs = [
        pl.BlockSpec((NTB, T_IN), lambda i: (i, 0)),
        pl.BlockSpec((NTB, T_IN), lambda i: (i, 0)),
    ]
    weights = (W0, W1, b0r, b1r, row2(bn_g), row2(bn_b), centers, proj_W,
               row2(proj_b), expWt, exp_b, row2(moe_lg), row2(moe_lb),
               row2(moe_rg), row2(moe_rb), qW, row2(qb), kW, row2(kb),
               vW, row2(vb), oW, row2(ob), row2(att_lg), row2(att_lb),
               row2(att_rg), row2(att_rb), headWl, headWr, row2(headb),
               jnp.where(jnp.equal(jnp.arange(NTB)[:, None] // CH_HALF,
                                   jnp.arange(NTB)[None, :] // CH_HALF),
                         0.0, -1e30).astype(jnp.float32))
    in_specs += [full(a) for a in weights]

    out = pl.pallas_call(
        _body,
        grid=grid,
        in_specs=in_specs,
        out_specs=pl.BlockSpec((B_TILE, NCLS), lambda i: (i, 0)),
        out_shape=jax.ShapeDtypeStruct((BS, NCLS), jnp.float32),
        compiler_params=pltpu.CompilerParams(
            dimension_semantics=("arbitrary",)),
    )(xl2, xr2, *weights)
    return out


# speedup vs baseline: 2.0353x; 2.0353x over previous
import functools

import jax
import jax.numpy as jnp
from jax import lax
from jax.experimental import pallas as pl
from jax.experimental.pallas import tpu as pltpu

BS, CH, T_IN, T_OUT, EXP_DIM, TOP_K, NCLS = 4096, 64, 128, 64, 32, 2, 3
CH_HALF = CH // 2

B_TILE = 32
NTB = B_TILE * CH_HALF
E_CHUNK = 16


def _ln(x, g, b, eps=1e-5):
    mu = jnp.mean(x, axis=-1, keepdims=True)
    xc = x - mu
    var = jnp.mean(xc * xc, axis=-1, keepdims=True)
    return xc / jnp.sqrt(var + eps) * g + b


def _bdot(a, b):
    return jnp.dot(a.astype(jnp.bfloat16), b.astype(jnp.bfloat16),
                   preferred_element_type=jnp.float32)


def _body(xl_ref, xr_ref, W0_ref, W1_ref, b0_ref, b1_ref, bn_g_ref, bn_b_ref,
          centers_ref, proj_W_ref, proj_b_ref, expWt_ref, exp_b_ref,
          moe_lg_ref, moe_lb_ref, moe_rg_ref, moe_rb_ref,
          qW_ref, qb_ref, kW_ref, kb_ref, vW_ref, vb_ref, oW_ref, ob_ref,
          att_lg_ref, att_lb_ref, att_rg_ref, att_rb_ref,
          headWl_ref, headWr_ref, headb_ref, emat_ref, out_ref):
    f32 = jnp.float32

    W0 = W0_ref[...]
    W1 = W1_ref[...]
    b0 = b0_ref[...]
    b1 = b1_ref[...]
    bn_g = bn_g_ref[...]
    bn_b = bn_b_ref[...]
    xl = xl_ref[...]
    xr = xr_ref[...]
    hl = jnp.maximum((_bdot(xl, W0) - b0) + (_bdot(xl, W1) - b1), 0.0)
    hr = jnp.maximum((_bdot(xr, W0) - b0) + (_bdot(xr, W1) - b1), 0.0)
    g_l = _ln(hl, bn_g, bn_b)
    g_r = _ln(hr, bn_g, bn_b)

    xcat = jnp.concatenate([g_l, g_r], axis=1)
    pr = _bdot(xcat, proj_W_ref[...]) + proj_b_ref[...]
    xp = pr / jnp.maximum(jnp.sqrt(jnp.sum(pr * pr, axis=-1, keepdims=True)),
                          1e-12)
    cn = centers_ref[...]
    cn = cn / jnp.maximum(jnp.sqrt(jnp.sum(cn * cn, axis=-1, keepdims=True)),
                          1e-12)
    sim = lax.dot_general(xp.astype(jnp.bfloat16), cn.astype(jnp.bfloat16),
                          (((1,), (1,)), ((), ())),
                          preferred_element_type=f32)

    eidx = lax.broadcasted_iota(jnp.int32, sim.shape, 1)
    m1 = jnp.max(sim, axis=-1, keepdims=True)
    i1 = jnp.min(jnp.where(sim == m1, eidx, CH), axis=-1, keepdims=True)
    sim2 = jnp.where(eidx == i1, -jnp.inf, sim)
    m2 = jnp.max(sim2, axis=-1, keepdims=True)
    i2 = jnp.min(jnp.where(sim2 == m2, eidx, CH), axis=-1, keepdims=True)
    w = (jnp.where(eidx == i1, 1.0, 0.0)
         + jnp.where(eidx == i2, 1.0, 0.0))

    gb_l = g_l.astype(jnp.bfloat16)
    gb_r = g_r.astype(jnp.bfloat16)
    wb = w.astype(jnp.bfloat16)
    Wt = expWt_ref[...].astype(jnp.bfloat16)
    Em = emat_ref[...].astype(jnp.bfloat16)
    acc_l = jnp.dot(w, exp_b_ref[...], preferred_element_type=f32)
    acc_r = acc_l
    for ec in range(0, CH, E_CHUNK):
        lo, hi = ec * T_OUT, (ec + E_CHUNK) * T_OUT
        Wc = Wt[:, lo:hi]
        ww = jnp.dot(wb, Em[:, lo:hi], preferred_element_type=f32)
        Tl = jnp.dot(gb_l, Wc, preferred_element_type=f32) * ww
        Tr = jnp.dot(gb_r, Wc, preferred_element_type=f32) * ww
        width = E_CHUNK * T_OUT
        while width > T_OUT:
            width //= 2
            Tl = Tl[:, :width] + Tl[:, width:]
            Tr = Tr[:, :width] + Tr[:, width:]
        acc_l = acc_l + Tl
        acc_r = acc_r + Tr
    m_l = _ln(acc_l, moe_lg_ref[...], moe_lb_ref[...]) + g_l
    m_r = _ln(acc_r, moe_rg_ref[...], moe_rb_ref[...]) + g_r

    q = _bdot(m_l, qW_ref[...]) + qb_ref[...]
    k = _bdot(m_r, kW_ref[...]) + kb_ref[...]
    v = _bdot(m_l - m_r, vW_ref[...]) + vb_ref[...]
    q3 = q.reshape(B_TILE, CH_HALF, T_OUT).astype(jnp.bfloat16)
    k3 = k.reshape(B_TILE, CH_HALF, T_OUT).astype(jnp.bfloat16)
    v3 = v.reshape(B_TILE, CH_HALF, T_OUT).astype(jnp.bfloat16)
    energy = lax.dot_general(q3, k3, (((2,), (2,)), ((0,), (0,))),
                             preferred_element_type=f32)
    energy = energy * (1.0 / (T_OUT ** 0.5))
    emax = jnp.max(energy, axis=-1, keepdims=True)
    ee = jnp.exp(energy - emax)
    attn = (ee / jnp.sum(ee, axis=-1, keepdims=True)).astype(jnp.bfloat16)
    o_l3 = lax.dot_general(attn, v3, (((2,), (1,)), ((0,), (0,))),
                           preferred_element_type=f32)
    o_r3 = lax.dot_general(attn, v3, (((1,), (1,)), ((0,), (0,))),
                           preferred_element_type=f32)
    o_l = o_l3.reshape(NTB, T_OUT)
    o_r = o_r3.reshape(NTB, T_OUT)
    oW = oW_ref[...]
    ob = ob_ref[...]
    f_l = _ln(_bdot(o_l, oW) + ob, att_lg_ref[...], att_lb_ref[...]) + m_l
    f_r = _ln(_bdot(o_r, oW) + ob, att_rg_ref[...], att_rb_ref[...]) + m_r

    fl3 = f_l.reshape(B_TILE, CH_HALF, T_OUT)
    fr3 = f_r.reshape(B_TILE, CH_HALF, T_OUT)
    Hl = headWl_ref[...]
    Hr = headWr_ref[...]
    cols = []
    for j in range(NCLS):
        s = (jnp.sum(fl3 * Hl[:, :, j][None], axis=(1, 2))
             + jnp.sum(fr3 * Hr[:, :, j][None], axis=(1, 2)))
        cols.append(s[:, None])
    out_ref[...] = jnp.concatenate(cols, axis=1) + headb_ref[...]


@jax.jit
def kernel(x_l, x_r, A, W0, b0, W1, b1, bn_g, bn_b, centers, proj_W, proj_b,
           exp_W, exp_b, moe_lg, moe_lb, moe_rg, moe_rb, qW, qb, kW, kb,
           vW, vb, oW, ob, att_lg, att_lb, att_rg, att_rb, headW, headb):
    del A
    xl2 = x_l.reshape(BS * CH_HALF, T_IN)
    xr2 = x_r.reshape(BS * CH_HALF, T_IN)
    b0r = b0.reshape(1, T_OUT)
    b1r = b1.reshape(1, T_OUT)
    expWt = exp_W.transpose(1, 0, 2).reshape(T_OUT, CH * T_OUT)
    headWl = headW[:CH_HALF * T_OUT].reshape(CH_HALF, T_OUT, NCLS)
    headWr = headW[CH_HALF * T_OUT:].reshape(CH_HALF, T_OUT, NCLS)
    row2 = lambda a: a.reshape(1, -1)

    grid = (BS // B_TILE,)
    full = lambda a: pl.BlockSpec(a.shape, lambda i: (0,) * a.ndim)
    in_specs = [
        pl.BlockSpec((NTB, T_IN), lambda i: (i, 0)),
        pl.BlockSpec((NTB, T_IN), lambda i: (i, 0)),
    ]
    weights = (W0, W1, b0r, b1r, row2(bn_g), row2(bn_b), centers, proj_W,
               row2(proj_b), expWt, exp_b, row2(moe_lg), row2(moe_lb),
               row2(moe_rg), row2(moe_rb), qW, row2(qb), kW, row2(kb),
               vW, row2(vb), oW, row2(ob), row2(att_lg), row2(att_lb),
               row2(att_rg), row2(att_rb), headWl, headWr, row2(headb),
               jnp.kron(jnp.eye(CH, dtype=jnp.float32),
                        jnp.ones((1, T_OUT), dtype=jnp.float32)))
    in_specs += [full(a) for a in weights]

    out = pl.pallas_call(
        _body,
        grid=grid,
        in_specs=in_specs,
        out_specs=pl.BlockSpec((B_TILE, NCLS), lambda i: (i, 0)),
        out_shape=jax.ShapeDtypeStruct((BS, NCLS), jnp.float32),
        compiler_params=pltpu.CompilerParams(
            dimension_semantics=("arbitrary",)),
    )(xl2, xr2, *weights)
    return out
